# Initial kernel scaffold; baseline (speedup 1.0000x reference)
import jax
import jax.numpy as jnp
from jax.experimental import pallas as pl


def kernel(x, W1, b1, W2, b2):
    raise NotImplementedError("write your pallas kernel here")


# speedup vs baseline: 1.3042x; 1.3042x over previous
import functools

import jax
import jax.numpy as jnp
from jax import lax
from jax.experimental import pallas as pl
from jax.experimental.pallas import tpu as pltpu
from jax.experimental.pallas import tpu_sc as plsc


def _router_body(x_ref, w1_ref, b1_ref, w2_ref, b2_ref, s_ref):
    T, C, N = x_ref.shape[1], x_ref.shape[2], x_ref.shape[3]
    xm = jnp.mean(x_ref[0], axis=0)

    iot = lax.broadcasted_iota(jnp.int32, (1, N), 1)
    hh = iot // 64
    ww = iot % 64

    acc = jnp.broadcast_to(b1_ref[0][:, None], (24, N))
    for k9 in range(9):
        dy, dx = k9 // 3 - 1, k9 % 3 - 1
        sft = dy * 64 + dx
        sh = jnp.roll(xm, -sft, axis=1) if sft else xm
        valid = ((hh + dy >= 0) & (hh + dy < 64)
                 & (ww + dx >= 0) & (ww + dx < 64))
        m = valid.astype(jnp.float32)
        acc = acc + lax.dot(w1_ref[k9], sh,
                            preferred_element_type=jnp.float32) * m

    h = jnp.where(acc >= 0, acc, 0.01 * acc)
    s = lax.dot(w2_ref[...], h, preferred_element_type=jnp.float32)
    s = s + b2_ref[0, 0]
    s_ref[0] = jax.nn.sigmoid(s)


def _router(x4, w1r, b1r, w2r, b2r):
    B, T, C, N = x4.shape
    return pl.pallas_call(
        _router_body,
        grid=(B,),
        in_specs=[
            pl.BlockSpec((1, T, C, N), lambda b: (b, 0, 0, 0)),
            pl.BlockSpec((9, 24, C), lambda b: (0, 0, 0)),
            pl.BlockSpec((1, 24), lambda b: (0, 0)),
            pl.BlockSpec((1, 24), lambda b: (0, 0)),
            pl.BlockSpec((1, 1), lambda b: (0, 0)),
        ],
        out_specs=pl.BlockSpec((1, 1, N), lambda b: (b, 0, 0)),
        out_shape=jax.ShapeDtypeStruct((B, 1, N), jnp.float32),
    )(x4, w1r, b1r, w2r, b2r).reshape(B, N)


def _make_sc_gather(B, P, N, K):
    info = plsc.get_sparse_core_info()
    NC, NS = info.num_cores, info.num_subcores
    NW = NC * NS
    per_w = P // NW
    mesh = plsc.VectorSubcoreMesh(core_axis_name="c", subcore_axis_name="s")

    @functools.partial(
        pl.kernel,
        mesh=mesh,
        compiler_params=pltpu.CompilerParams(needs_layout_passes=False),
        out_type=jax.ShapeDtypeStruct((B * P, K), jnp.float32),
        scratch_types=[
            pltpu.VMEM((K,), jnp.int32),
            pltpu.VMEM((N,), jnp.float32),
            pltpu.VMEM((K,), jnp.float32),
        ],
    )
    def gather_k(x_hbm, idx_hbm, y_hbm, idx_v, plane_v, out_v):
        wid = lax.axis_index("s") * NC + lax.axis_index("c")
        for b in range(B):
            pltpu.sync_copy(idx_hbm.at[b], idx_v)
            base = b * P + wid * per_w

            def plane_body(p, _):
                row = base + p
                pltpu.sync_copy(x_hbm.at[row], plane_v)

                def g_body(j, carry):
                    iv = idx_v[pl.ds(j * 16, 16)]
                    out_v[pl.ds(j * 16, 16)] = plsc.load_gather(
                        plane_v, [iv])
                    return carry

                lax.fori_loop(0, K // 16, g_body, 0)
                pltpu.sync_copy(out_v, y_hbm.at[row])
                return _

            lax.fori_loop(0, per_w, plane_body, 0)

    return gather_k


def _tr_body(y_ref, sc_ref, o_ref):
    o_ref[0] = jnp.transpose(y_ref[0]) * sc_ref[0, 0][:, None]


def _transpose_scale(y3, scale):
    B, P, K = y3.shape
    BK, BP = 256, 256
    return pl.pallas_call(
        _tr_body,
        grid=(B, K // BK, P // BP),
        in_specs=[
            pl.BlockSpec((1, BP, BK), lambda b, ki, pj: (b, pj, ki)),
            pl.BlockSpec((1, 1, BK), lambda b, ki, pj: (b, 0, ki)),
        ],
        out_specs=pl.BlockSpec((1, BK, BP), lambda b, ki, pj: (b, ki, pj)),
        out_shape=jax.ShapeDtypeStruct((B, K, P), jnp.float32),
    )(y3, scale.reshape(B, 1, K))


def kernel(x, W1, b1, W2, b2):
    B, T, C, H, W = x.shape
    N = H * W
    K = max(1, N // 2)
    P = T * C

    x4 = x.reshape(B, T, C, N)
    w1r = jnp.transpose(W1, (2, 3, 0, 1)).reshape(9, 24, C)
    b1r = b1.reshape(1, 24)
    w2r = W2.reshape(1, 24)
    b2r = b2.reshape(1, 1)

    scores = _router(x4, w1r, b1r, w2r, b2r)

    _, top_i = lax.top_k(scores, K)
    idx = jnp.sort(top_i, axis=1).astype(jnp.int32)
    sg = jnp.take_along_axis(scores, idx, axis=1)
    scale = sg / (sg + 1e-6)

    y2 = _make_sc_gather(B, P, N, K)(x4.reshape(B * P, N), idx)
    out = _transpose_scale(y2.reshape(B, P, K), scale)
    return out.reshape(B, K, T, C), idx
